# baseline (speedup 1.0000x reference)
import functools
import math

import jax
import jax.numpy as jnp
from jax import lax
from jax.experimental import pallas as pl
from jax.experimental.pallas import tpu as pltpu
from jax.experimental.pallas import tpu_sc as plsc

_DIM = 64
_SCALE = math.sqrt(_DIM)
_LANES = 16
_VBLK = 2048
_CHUNK = 256


@functools.lru_cache(maxsize=None)
def _build_prep(vocab: int, dim: int):
    grid = (vocab + _VBLK - 1) // _VBLK

    def body(t_ref, o_ref):
        t = t_ref[...].astype(jnp.float32) * _SCALE
        o_ref[...] = jnp.pad(t.T, ((0, 0), (0, 128 - dim)))

    return pl.pallas_call(
        body,
        grid=(grid,),
        in_specs=[pl.BlockSpec((dim, _VBLK), lambda i: (0, i))],
        out_specs=pl.BlockSpec((_VBLK, 128), lambda i: (i, 0)),
        out_shape=jax.ShapeDtypeStruct((vocab, 128), jnp.float32),
        compiler_params=pltpu.CompilerParams(
            dimension_semantics=("arbitrary",)
        ),
    )


@functools.lru_cache(maxsize=None)
def _build_gather(seq: int, batch: int, vocab: int, dim: int):
    info = plsc.get_sparse_core_info()
    nw = info.num_cores * info.num_subcores
    chunks_per_s = batch // _CHUNK
    n_units = seq * chunks_per_s
    assert n_units % nw == 0
    units_per_w = n_units // nw

    mesh = plsc.VectorSubcoreMesh(core_axis_name="c", subcore_axis_name="s")

    @functools.partial(
        pl.kernel,
        mesh=mesh,
        compiler_params=pltpu.CompilerParams(needs_layout_passes=False),
        out_type=jax.ShapeDtypeStruct((seq, dim, batch), jnp.float32),
        scratch_types=[
            pltpu.VMEM((_CHUNK,), jnp.int32),
            pltpu.VMEM((_CHUNK,), jnp.int32),
            pltpu.VMEM((_CHUNK, 128), jnp.float32),
            pltpu.VMEM((_CHUNK, 128), jnp.float32),
            pltpu.VMEM((dim, _CHUNK), jnp.float32),
            pltpu.VMEM((dim, _CHUNK), jnp.float32),
            pltpu.SemaphoreType.DMA,
            pltpu.SemaphoreType.DMA,
            pltpu.SemaphoreType.DMA,
            pltpu.SemaphoreType.DMA,
        ],
    )
    def sc_embed(xt_hbm, tab_hbm, out_hbm, i0, i1, g0, g1, o0, o1,
                 gs0, gs1, ss0, ss1):
        wid = lax.axis_index("s") * info.num_cores + lax.axis_index("c")
        u_base = wid * units_per_w
        idxs = (i0, i1)
        gbufs = (g0, g1)
        obufs = (o0, o1)
        gsems = (gs0, gs1)
        ssems = (ss0, ss1)

        lane = lax.iota(jnp.int32, _LANES)

        def start_gather(k):
            u = u_base + k
            s = u // chunks_per_s
            b0 = (u % chunks_per_s) * _CHUNK
            p = k % 2
            pltpu.sync_copy(xt_hbm.at[s, pl.ds(b0, _CHUNK)], idxs[p])
            return pltpu.async_copy(tab_hbm.at[idxs[p]], gbufs[p], gsems[p])

        def transpose_block(g, o):
            def col_group(gi, carry):
                rows = gi * _LANES + lane

                def per_d(d, c2):
                    vals = plsc.load_gather(g, [rows, jnp.full((_LANES,), d, jnp.int32)])
                    o[d, pl.ds(gi * _LANES, _LANES)] = vals
                    return c2

                return lax.fori_loop(0, dim, per_d, carry)

            lax.fori_loop(0, _CHUNK // _LANES, col_group, 0)

        def start_store(k):
            u = u_base + k
            s = u // chunks_per_s
            b0 = (u % chunks_per_s) * _CHUNK
            p = k % 2
            return pltpu.async_copy(
                obufs[p], out_hbm.at[s, :, pl.ds(b0, _CHUNK)], ssems[p]
            )

        gathers = [None, None]
        stores = [None, None]
        gathers[0] = start_gather(0)
        for k in range(units_per_w):
            p = k % 2
            if k + 1 < units_per_w:
                if k >= 1:
                    stores[(k + 1) % 2].wait()
                gathers[(k + 1) % 2] = start_gather(k + 1)
            gathers[p].wait()
            transpose_block(gbufs[p], obufs[p])
            stores[p] = start_store(k)
        stores[units_per_w % 2].wait()
        stores[(units_per_w + 1) % 2].wait()

    return sc_embed


def kernel(x, table):
    b, s = x.shape
    vocab, dim = table.shape
    tab_p = _build_prep(vocab, dim)(table.T)
    out_t = _build_gather(s, b, vocab, dim)(
        x.T.astype(jnp.int32), tab_p
    )
    return out_t.transpose(2, 0, 1)
